# baseline (speedup 1.0000x reference)
import functools

import jax
import jax.numpy as jnp
from jax.experimental import pallas as pl
from jax.experimental.pallas import tpu as pltpu


def _body(x_ref, w_ref, el_ref, diff_ref, lsn_ref, out_ref, loss_ref, *, M, N):
    for n in range(N):
        acc = w_ref[:, 0, n : n + 1] * x_ref[:, 0, :]
        for m in range(1, M):
            acc = acc + w_ref[:, m, n : n + 1] * x_ref[:, m, :]
        out_ref[:, n, :] = acc
    for n in range(N):
        col = el_ref[:, n : n + 1]
        for m in range(M):
            col = col + (w_ref[:, m, n : n + 1] * diff_ref[m, n] + lsn_ref[m, n])
        loss_ref[:, n : n + 1] = col


def kernel(x, extra_loss, logits):
    B, M, F = x.shape
    N = logits.shape[1]
    probs = jax.nn.sigmoid(logits)
    w = jax.random.bernoulli(jax.random.key(42), probs, shape=(B, M, N)).astype(
        jnp.float32
    )
    ls = jax.nn.log_sigmoid(logits)
    lsn = jax.nn.log_sigmoid(-logits)
    diff = ls - lsn

    bB = 256
    grid = (B // bB,)
    out_shapes = (
        jax.ShapeDtypeStruct((B, N, F), jnp.float32),
        jax.ShapeDtypeStruct((B, N), jnp.float32),
    )
    fn = pl.pallas_call(
        functools.partial(_body, M=M, N=N),
        grid=grid,
        in_specs=[
            pl.BlockSpec((bB, M, F), lambda i: (i, 0, 0)),
            pl.BlockSpec((bB, M, N), lambda i: (i, 0, 0)),
            pl.BlockSpec((bB, N), lambda i: (i, 0)),
            pl.BlockSpec(memory_space=pltpu.SMEM),
            pl.BlockSpec(memory_space=pltpu.SMEM),
        ],
        out_specs=(
            pl.BlockSpec((bB, N, F), lambda i: (i, 0, 0)),
            pl.BlockSpec((bB, N), lambda i: (i, 0)),
        ),
        out_shape=out_shapes,
        compiler_params=pltpu.CompilerParams(
            dimension_semantics=("arbitrary",),
        ),
    )
    out, loss = fn(x, w, extra_loss, diff, lsn)
    return (out, loss)


# speedup vs baseline: 1.4262x; 1.4262x over previous
import functools

import jax
import jax.numpy as jnp
from jax.experimental import pallas as pl
from jax.experimental.pallas import tpu as pltpu


def _body(x_ref, w_ref, el_ref, diff_ref, lsn_ref, out_ref, loss_ref, *, M, N):
    x = x_ref[...]
    w = w_ref[...]
    for n in range(N):
        out_ref[:, n, :] = jnp.sum(w[:, :, n : n + 1] * x, axis=1)
    for n in range(N):
        col = el_ref[:, n : n + 1]
        for m in range(M):
            col = col + (w_ref[:, m, n : n + 1] * diff_ref[m, n] + lsn_ref[m, n])
        loss_ref[:, n : n + 1] = col


def kernel(x, extra_loss, logits):
    B, M, F = x.shape
    N = logits.shape[1]
    probs = jax.nn.sigmoid(logits)
    w = jax.random.bernoulli(jax.random.key(42), probs, shape=(B, M, N)).astype(
        jnp.float32
    )
    ls = jax.nn.log_sigmoid(logits)
    lsn = jax.nn.log_sigmoid(-logits)
    diff = ls - lsn

    bB = 256
    grid = (B // bB,)
    out_shapes = (
        jax.ShapeDtypeStruct((B, N, F), jnp.float32),
        jax.ShapeDtypeStruct((B, N), jnp.float32),
    )
    fn = pl.pallas_call(
        functools.partial(_body, M=M, N=N),
        grid=grid,
        in_specs=[
            pl.BlockSpec((bB, M, F), lambda i: (i, 0, 0)),
            pl.BlockSpec((bB, M, N), lambda i: (i, 0, 0)),
            pl.BlockSpec((bB, N), lambda i: (i, 0)),
            pl.BlockSpec(memory_space=pltpu.SMEM),
            pl.BlockSpec(memory_space=pltpu.SMEM),
        ],
        out_specs=(
            pl.BlockSpec((bB, N, F), lambda i: (i, 0, 0)),
            pl.BlockSpec((bB, N), lambda i: (i, 0)),
        ),
        out_shape=out_shapes,
        compiler_params=pltpu.CompilerParams(
            dimension_semantics=("arbitrary",),
        ),
    )
    out, loss = fn(x, w, extra_loss, diff, lsn)
    return (out, loss)
